# Initial kernel scaffold; baseline (speedup 1.0000x reference)
import jax
import jax.numpy as jnp
from jax.experimental import pallas as pl


def kernel(x, pair_basis, triplet_basis, idx_kj, idx_ji, W_pb, W_tb, W_kj, b_kj, W_ji, b_ji, W_down, W_up, res_b_W1, res_b_b1, res_b_W2, res_b_b2, res_a_W1, res_a_b1, res_a_W2, res_a_b2):
    raise NotImplementedError("write your pallas kernel here")


# speedup vs baseline: 1.2257x; 1.2257x over previous
import functools

import jax
import jax.numpy as jnp
from jax import lax
from jax.experimental import pallas as pl
from jax.experimental.pallas import tpu as pltpu
from jax.experimental.pallas import tpu_sc as plsc

E = 320000
T = 640000
H = 128
I = 64

F32 = jnp.float32


def _silu(v):
    return v / (1.0 + jnp.exp(-v))


_BE = 1280


def _tc_a_body(x_ref, pbas_ref, Wpb, Wji, bji, Wkj, bkj, Wdn, xji_out, g_out):
    x = x_ref[...]
    pb = _silu(jnp.dot(pbas_ref[...], Wpb[...], preferred_element_type=F32))
    xji = _silu(jnp.dot(x, Wji[...], preferred_element_type=F32) + bji[...])
    xkj = _silu(jnp.dot(x, Wkj[...], preferred_element_type=F32) + bkj[...])
    xkj = xkj * pb
    g_out[...] = _silu(jnp.dot(xkj, Wdn[...], preferred_element_type=F32))
    xji_out[...] = xji


def _tc_a(x, pair_basis, W_pb, W_ji, b_ji, W_kj, b_kj, W_down):
    n = E // _BE
    row = lambda w: pl.BlockSpec((_BE, w), lambda i: (i, 0))
    full = lambda a, b: pl.BlockSpec((a, b), lambda i: (0, 0))
    return pl.pallas_call(
        _tc_a_body,
        grid=(n,),
        in_specs=[
            row(H), row(16),
            full(16, H), full(H, H), full(1, H), full(H, H), full(1, H),
            full(H, I),
        ],
        out_specs=[row(H), row(I)],
        out_shape=[
            jax.ShapeDtypeStruct((E, H), F32),
            jax.ShapeDtypeStruct((E, I), F32),
        ],
        compiler_params=pltpu.CompilerParams(
            dimension_semantics=("arbitrary",)),
    )(x, pair_basis, W_pb, W_ji, b_ji.reshape(1, H), W_kj, b_kj.reshape(1, H),
      W_down)


_BT = 2560


def _tc_b_body(tbas_ref, Wtb, tb_out):
    tb_out[...] = _silu(
        jnp.dot(tbas_ref[...], Wtb[...], preferred_element_type=F32))


def _tc_b(triplet_basis, W_tb):
    n = T // _BT
    return pl.pallas_call(
        _tc_b_body,
        grid=(n,),
        in_specs=[
            pl.BlockSpec((_BT, 64), lambda i: (i, 0)),
            pl.BlockSpec((64, I), lambda i: (0, 0)),
        ],
        out_specs=pl.BlockSpec((_BT, I), lambda i: (i, 0)),
        out_shape=jax.ShapeDtypeStruct((T, I), F32),
        compiler_params=pltpu.CompilerParams(
            dimension_semantics=("arbitrary",)),
    )(triplet_basis, W_tb)


def _tc_c_body(seg_ref, xji_ref, x_ref, Wup,
               b1W1, b1b1, b1W2, b1b2,
               a1W1, a1b1, a1W2, a1b2,
               a2W1, a2b1, a2W2, a2b2,
               h_out):
    h = xji_ref[...] + _silu(
        jnp.dot(seg_ref[...], Wup[...], preferred_element_type=F32))

    def res(h, W1, b1, W2, b2):
        u = _silu(jnp.dot(h, W1[...], preferred_element_type=F32) + b1[...])
        return h + _silu(jnp.dot(u, W2[...], preferred_element_type=F32)
                         + b2[...])

    h = res(h, b1W1, b1b1, b1W2, b1b2)
    h = h + x_ref[...]
    h = res(h, a1W1, a1b1, a1W2, a1b2)
    h = res(h, a2W1, a2b1, a2W2, a2b2)
    h_out[...] = h


def _tc_c(seg, x_ji, x, W_up, res_b_W1, res_b_b1, res_b_W2, res_b_b2,
          res_a_W1, res_a_b1, res_a_W2, res_a_b2):
    n = E // _BE
    row = lambda w: pl.BlockSpec((_BE, w), lambda i: (i, 0))
    full = lambda a, b: pl.BlockSpec((a, b), lambda i: (0, 0))
    wspecs = []
    wargs = []
    for l, (W1, bb1, W2, bb2) in enumerate(
            [(res_b_W1[0], res_b_b1[0], res_b_W2[0], res_b_b2[0]),
             (res_a_W1[0], res_a_b1[0], res_a_W2[0], res_a_b2[0]),
             (res_a_W1[1], res_a_b1[1], res_a_W2[1], res_a_b2[1])]):
        wargs += [W1, bb1.reshape(1, H), W2, bb2.reshape(1, H)]
        wspecs += [full(H, H), full(1, H), full(H, H), full(1, H)]
    return pl.pallas_call(
        _tc_c_body,
        grid=(n,),
        in_specs=[row(I), row(H), row(H), full(I, H)] + wspecs,
        out_specs=row(H),
        out_shape=jax.ShapeDtypeStruct((E, H), F32),
        compiler_params=pltpu.CompilerParams(
            dimension_semantics=("arbitrary",)),
    )(seg, x_ji, x, W_up, *wargs)


_NC = 2
_NS = 16
_NP = 10
_DC = 16000
_PAD = 384
_ACCR = _DC + _PAD
_TW = T // _NS
_S = 8000
_NSEG = _TW // _S
_NV = _S // 16
_BLK = 128


def _sc_body(g_hbm, tb_hbm, kj_hbm, ji_hbm, out_hbm,
             ji_v, kj_v, tc_c, kc_c, jc_c,
             t_blk, kj_blk, ji_blk, tb_rows, g_rows, zrows,
             acc, sem0, sem1):
    c = lax.axis_index("c")
    s = lax.axis_index("s")
    lanes = lax.iota(jnp.int32, 16)
    zero16f = jnp.zeros((16,), F32)

    def zinit(r, _):
        for j in range(4):
            zrows[r, pl.ds(j * 16, 16)] = zero16f
        return 0
    lax.fori_loop(0, _BLK, zinit, 0)

    def one_pass(p, _):
        chunk_lo = (c * _NP + p) * _DC

        rows_per_tile = _ACCR // _NS
        def zblk(z, _):
            pltpu.sync_copy(
                zrows, acc.at[pl.ds(s * rows_per_tile + z * _BLK, _BLK)])
            return 0
        lax.fori_loop(0, rows_per_tile // _BLK, zblk, 0)
        plsc.subcore_barrier()

        def segment(seg, _):
            base_t = s * _TW + seg * _S
            pltpu.sync_copy(ji_hbm.at[pl.ds(base_t, _S)], ji_v)
            pltpu.sync_copy(kj_hbm.at[pl.ds(base_t, _S)], kj_v)

            def scank(k, off):
                ji16 = ji_v[pl.ds(k * 16, 16)]
                kj16 = kj_v[pl.ds(k * 16, 16)]
                jil = ji16 - chunk_lo
                m = (jil >= 0) & (jil < _DC)
                t16 = base_t + k * 16 + lanes
                ones = jnp.where(m, 1, 0).astype(jnp.int32)
                pos = off + plsc.cumsum(ones) - 1
                plsc.store_scatter(tc_c, [pos], t16, mask=m)
                plsc.store_scatter(kc_c, [pos], kj16, mask=m)
                plsc.store_scatter(jc_c, [pos], jil, mask=m)
                return off + jnp.sum(ones)

            off = lax.fori_loop(0, _NV, scank, jnp.int32(0))

            for j in range(_BLK // 16):
                dvec = lanes * 16 + j
                tc_c[pl.ds(off + j * 16, 16)] = dvec
                kc_c[pl.ds(off + j * 16, 16)] = dvec
                jc_c[pl.ds(off + j * 16, 16)] = _DC + j * 16 + lanes

            nblk = (off + (_BLK - 1)) // _BLK

            def flush(b, _):
                for j in range(_BLK // 16):
                    sl_src = pl.ds(b * _BLK + j * 16, 16)
                    sl_dst = pl.ds(j * 16, 16)
                    t_blk[sl_dst] = tc_c[sl_src]
                    kj_blk[sl_dst] = kc_c[sl_src]
                    ji_blk[sl_dst] = jc_c[sl_src]
                d0 = pltpu.async_copy(tb_hbm.at[t_blk], tb_rows, sem0)
                d1 = pltpu.async_copy(g_hbm.at[kj_blk], g_rows, sem1)
                d0.wait()
                d1.wait()

                def mulr(r, _):
                    for j in range(4):
                        sl = pl.ds(j * 16, 16)
                        g_rows[r, sl] = g_rows[r, sl] * tb_rows[r, sl]
                    return 0
                lax.fori_loop(0, _BLK, mulr, 0)
                pltpu.sync_copy(g_rows, acc.at[ji_blk], add=True)
                return 0

            lax.fori_loop(0, nblk, flush, 0)
            return 0

        lax.fori_loop(0, _NSEG, segment, 0)
        plsc.subcore_barrier()

        wrows = _DC // _NS
        pltpu.sync_copy(acc.at[pl.ds(s * wrows, wrows)],
                        out_hbm.at[pl.ds(chunk_lo + s * wrows, wrows)])
        plsc.subcore_barrier()
        return 0

    lax.fori_loop(0, _NP, one_pass, 0)


def _sc_segment_sum(g, tb, idx_kj, idx_ji):
    mesh = plsc.VectorSubcoreMesh(
        core_axis_name="c", subcore_axis_name="s",
        num_cores=_NC, num_subcores=_NS)
    kern = pl.kernel(
        _sc_body,
        out_type=jax.ShapeDtypeStruct((E, I), F32),
        mesh=mesh,
        compiler_params=pltpu.CompilerParams(
            needs_layout_passes=False, use_tc_tiling_on_sc=False),
        scratch_types=[
            pltpu.VMEM((_S,), jnp.int32),
            pltpu.VMEM((_S,), jnp.int32),
            pltpu.VMEM((_S + _BLK,), jnp.int32),
            pltpu.VMEM((_S + _BLK,), jnp.int32),
            pltpu.VMEM((_S + _BLK,), jnp.int32),
            pltpu.VMEM((_BLK,), jnp.int32),
            pltpu.VMEM((_BLK,), jnp.int32),
            pltpu.VMEM((_BLK,), jnp.int32),
            pltpu.VMEM((_BLK, I), F32),
            pltpu.VMEM((_BLK, I), F32),
            pltpu.VMEM((_BLK, I), F32),
            pltpu.MemorySpace.VMEM_SHARED((_ACCR, I), F32),
            pltpu.SemaphoreType.DMA,
            pltpu.SemaphoreType.DMA,
        ],
    )
    return kern(g, tb, idx_kj, idx_ji)


def kernel(x, pair_basis, triplet_basis, idx_kj, idx_ji, W_pb, W_tb, W_kj,
           b_kj, W_ji, b_ji, W_down, W_up, res_b_W1, res_b_b1, res_b_W2,
           res_b_b2, res_a_W1, res_a_b1, res_a_W2, res_a_b2):
    x_ji, g = _tc_a(x, pair_basis, W_pb, W_ji, b_ji, W_kj, b_kj, W_down)
    tb = _tc_b(triplet_basis, W_tb)
    seg = _sc_segment_sum(g, tb, idx_kj, idx_ji)
    return _tc_c(seg, x_ji, x, W_up, res_b_W1, res_b_b1, res_b_W2, res_b_b2,
                 res_a_W1, res_a_b1, res_a_W2, res_a_b2)
